# Initial kernel scaffold; baseline (speedup 1.0000x reference)
import jax
import jax.numpy as jnp
from jax.experimental import pallas as pl


def kernel(input_bits, connections, memory):
    raise NotImplementedError("write your pallas kernel here")


# speedup vs baseline: 2.2395x; 2.2395x over previous
import functools

import jax
import jax.numpy as jnp
from jax import lax
from jax.experimental import pallas as pl
from jax.experimental.pallas import tpu as pltpu
from jax.experimental.pallas import tpu_sc as plsc

_B = 4096
_TOTAL_BITS = 1024
_NUM_NEURONS = 1024
_N_BITS = 14
_MEM_SIZE = 1 << _N_BITS

_BM = 512
_NW = 32
_FLAT = _B * _NUM_NEURONS
_PER_W = _FLAT // _NW
_CH = 16384
_NCH = _PER_W // _CH


def _addr_body(bits_ref, connt_ref, out_ref, w_ref):
    @pl.when(pl.program_id(0) == 0)
    def _build_w():
        t = lax.broadcasted_iota(jnp.int32, (_TOTAL_BITS, _NUM_NEURONS), 0)
        w = jnp.zeros((_TOTAL_BITS, _NUM_NEURONS), jnp.float32)
        for k in range(_N_BITS):
            c = connt_ref[k : k + 1, :]
            w = w + jnp.where(t == c, jnp.float32(1 << (_N_BITS - 1 - k)), 0.0)
        w_ref[:] = w

    bits = (bits_ref[:] != 0).astype(jnp.float32)
    addr = jnp.dot(bits, w_ref[:], preferred_element_type=jnp.float32,
                   precision=lax.Precision.HIGHEST)
    col = lax.broadcasted_iota(jnp.int32, (_BM, _NUM_NEURONS), 1)
    out_ref[:] = addr.astype(jnp.int32) + col * _MEM_SIZE


def _addresses(input_bits, connt):
    return pl.pallas_call(
        _addr_body,
        grid=(_B // _BM,),
        in_specs=[
            pl.BlockSpec((_BM, _TOTAL_BITS), lambda i: (i, 0)),
            pl.BlockSpec((16, _NUM_NEURONS), lambda i: (0, 0)),
        ],
        out_specs=pl.BlockSpec((_BM, _NUM_NEURONS), lambda i: (i, 0)),
        out_shape=jax.ShapeDtypeStruct((_B, _NUM_NEURONS), jnp.int32),
        scratch_shapes=[pltpu.VMEM((_TOTAL_BITS, _NUM_NEURONS), jnp.float32)],
    )(input_bits, connt)


@functools.lru_cache(maxsize=1)
def _make_gather():
    mesh = plsc.VectorSubcoreMesh(core_axis_name="c", subcore_axis_name="s")

    @functools.partial(
        pl.kernel,
        mesh=mesh,
        out_type=jax.ShapeDtypeStruct((_FLAT,), jnp.float32),
        scratch_types=[
            pltpu.VMEM((_CH,), jnp.int32),
            pltpu.VMEM((_CH,), jnp.float32),
            pltpu.SemaphoreType.DMA,
        ],
    )
    def gather_k(mem_hbm, idx_hbm, out_hbm, idx_v, val_v, sem):
        wid = lax.axis_index("s") * 2 + lax.axis_index("c")
        base = wid * _PER_W
        for c in range(_NCH):
            off = base + c * _CH
            pltpu.sync_copy(idx_hbm.at[pl.ds(off, _CH)], idx_v)
            pltpu.async_copy(mem_hbm.at[idx_v], val_v, sem).wait()
            pltpu.sync_copy(val_v, out_hbm.at[pl.ds(off, _CH)])

    return gather_k


def kernel(input_bits, connections, memory):
    connt = jnp.full((16, _NUM_NEURONS), _TOTAL_BITS, jnp.int32)
    connt = connt.at[:_N_BITS].set(connections.T)
    idx = _addresses(input_bits, connt)
    out_flat = _make_gather()(memory.reshape(-1), idx.reshape(-1))
    return out_flat.reshape(_B, _NUM_NEURONS)


# speedup vs baseline: 3.0178x; 1.3475x over previous
import functools

import jax
import jax.numpy as jnp
from jax import lax
from jax.experimental import pallas as pl
from jax.experimental.pallas import tpu as pltpu
from jax.experimental.pallas import tpu_sc as plsc

_B = 4096
_TOTAL_BITS = 1024
_NUM_NEURONS = 1024
_N_BITS = 14
_MEM_SIZE = 1 << _N_BITS

_BM = 512
_NW = 32
_FLAT = _B * _NUM_NEURONS
_PER_W = _FLAT // _NW
_CH = 16384
_NCH = _PER_W // _CH


def _addr_body(bits_ref, connt_ref, out_ref, w_ref):
    @pl.when(pl.program_id(0) == 0)
    def _build_w():
        t = lax.broadcasted_iota(jnp.int32, (_TOTAL_BITS, _NUM_NEURONS), 0)
        w = jnp.zeros((_TOTAL_BITS, _NUM_NEURONS), jnp.float32)
        for k in range(_N_BITS):
            c = connt_ref[k : k + 1, :]
            w = w + jnp.where(t == c, jnp.float32(1 << (_N_BITS - 1 - k)), 0.0)
        w_ref[:] = w

    bits = (bits_ref[:] != 0).astype(jnp.float32)
    addr = jnp.dot(bits, w_ref[:], preferred_element_type=jnp.float32,
                   precision=lax.Precision.HIGHEST).astype(jnp.int32)
    n = lax.broadcasted_iota(jnp.int32, (_BM, _NUM_NEURONS), 1)
    phys = (
        (n >> 3) * (_MEM_SIZE * 8)
        + (addr >> 7) * 1024
        + (n & 7) * 128
        + (addr & 127)
    )
    out_ref[:] = (
        phys.reshape(_BM // 8, 8, _NUM_NEURONS // 128, 128)
        .transpose(0, 2, 1, 3)
        .reshape(_BM * _NUM_NEURONS)
    )


def _addresses(input_bits, connt):
    return pl.pallas_call(
        _addr_body,
        grid=(_B // _BM,),
        in_specs=[
            pl.BlockSpec((_BM, _TOTAL_BITS), lambda i: (i, 0)),
            pl.BlockSpec((16, _NUM_NEURONS), lambda i: (0, 0)),
        ],
        out_specs=pl.BlockSpec((_BM * _NUM_NEURONS,), lambda i: (i,)),
        out_shape=jax.ShapeDtypeStruct((_FLAT,), jnp.int32),
        scratch_shapes=[pltpu.VMEM((_TOTAL_BITS, _NUM_NEURONS), jnp.float32)],
    )(input_bits, connt)


@functools.lru_cache(maxsize=1)
def _make_gather():
    mesh = plsc.VectorSubcoreMesh(core_axis_name="c", subcore_axis_name="s")

    @functools.partial(
        pl.kernel,
        mesh=mesh,
        out_type=jax.ShapeDtypeStruct((_FLAT,), jnp.float32),
        scratch_types=[
            pltpu.VMEM((_CH,), jnp.int32),
            pltpu.VMEM((_CH,), jnp.int32),
            pltpu.VMEM((_CH,), jnp.float32),
            pltpu.VMEM((_CH,), jnp.float32),
            pltpu.SemaphoreType.DMA,
            pltpu.SemaphoreType.DMA,
        ],
    )
    def gather_k(mem_hbm, idx_hbm, out_hbm, idx_v0, idx_v1, val_v0, val_v1,
                 gsem, wsem):
        wid = lax.axis_index("s") * 2 + lax.axis_index("c")
        base = wid * _PER_W
        idx_bufs = (idx_v0, idx_v1)
        val_bufs = (val_v0, val_v1)

        pltpu.sync_copy(idx_hbm.at[pl.ds(base, _CH)], idx_v0)
        gathers = [None, None]
        wbs = [None, None]
        gathers[0] = pltpu.async_copy(mem_hbm.at[idx_v0], val_v0, gsem)
        for c in range(_NCH):
            cur = c % 2
            nxt = (c + 1) % 2
            if c + 1 < _NCH:
                pltpu.sync_copy(
                    idx_hbm.at[pl.ds(base + (c + 1) * _CH, _CH)], idx_bufs[nxt]
                )
            gathers[cur].wait()
            wbs[cur] = pltpu.async_copy(
                val_bufs[cur], out_hbm.at[pl.ds(base + c * _CH, _CH)], wsem
            )
            if c + 1 < _NCH:
                if wbs[nxt] is not None:
                    wbs[nxt].wait()
                gathers[nxt] = pltpu.async_copy(
                    mem_hbm.at[idx_bufs[nxt]], val_bufs[nxt], gsem
                )
        wbs[(_NCH - 1) % 2].wait()

    return gather_k


def kernel(input_bits, connections, memory):
    connt = jnp.full((16, _NUM_NEURONS), _TOTAL_BITS, jnp.int32)
    connt = connt.at[:_N_BITS].set(connections.T)
    idx = _addresses(input_bits, connt)
    mem_lin = (
        memory.reshape(_NUM_NEURONS // 8, 8, _MEM_SIZE // 128, 128)
        .transpose(0, 2, 1, 3)
        .reshape(_NUM_NEURONS * _MEM_SIZE)
    )
    out_flat = _make_gather()(mem_lin, idx)
    return (
        out_flat.reshape(_B // 8, 8, _NUM_NEURONS // 128, 128)
        .transpose(0, 2, 1, 3)
        .reshape(_B, _NUM_NEURONS)
    )


# speedup vs baseline: 3.0586x; 1.0135x over previous
import functools

import jax
import jax.numpy as jnp
from jax import lax
from jax.experimental import pallas as pl
from jax.experimental.pallas import tpu as pltpu
from jax.experimental.pallas import tpu_sc as plsc

_B = 4096
_TOTAL_BITS = 1024
_NUM_NEURONS = 1024
_N_BITS = 14
_MEM_SIZE = 1 << _N_BITS

_BM = 512
_NW = 32
_FLAT = _B * _NUM_NEURONS
_PER_W = _FLAT // _NW
_CH = 16384
_NCH = _PER_W // _CH


def _addr_body(bits_ref, connt_ref, out_ref, w_ref):
    @pl.when(pl.program_id(0) == 0)
    def _build_w():
        t = lax.broadcasted_iota(jnp.int32, (_TOTAL_BITS, _NUM_NEURONS), 0)
        w = jnp.zeros((_TOTAL_BITS, _NUM_NEURONS), jnp.float32)
        for k in range(_N_BITS):
            c = connt_ref[k : k + 1, :]
            w = w + jnp.where(t == c, jnp.float32(1 << (_N_BITS - 1 - k)), 0.0)
        w_ref[:] = w

    bits = (bits_ref[:] != 0).astype(jnp.float32)
    addr = jnp.dot(bits, w_ref[:], preferred_element_type=jnp.float32,
                   precision=lax.Precision.HIGHEST).astype(jnp.int32)
    n = lax.broadcasted_iota(jnp.int32, (_BM, _NUM_NEURONS), 1)
    phys = (
        (n >> 3) * (_MEM_SIZE * 8)
        + (addr >> 7) * 1024
        + (n & 7) * 128
        + (addr & 127)
    )
    out_ref[:] = (
        phys.reshape(_BM // 8, 8, _NUM_NEURONS // 128, 128)
        .transpose(0, 2, 1, 3)
        .reshape(_BM * _NUM_NEURONS)
    )


def _addresses(input_bits, connt):
    return pl.pallas_call(
        _addr_body,
        grid=(_B // _BM,),
        in_specs=[
            pl.BlockSpec((_BM, _TOTAL_BITS), lambda i: (i, 0)),
            pl.BlockSpec((16, _NUM_NEURONS), lambda i: (0, 0)),
        ],
        out_specs=pl.BlockSpec((_BM * _NUM_NEURONS,), lambda i: (i,)),
        out_shape=jax.ShapeDtypeStruct((_FLAT,), jnp.int32),
        scratch_shapes=[pltpu.VMEM((_TOTAL_BITS, _NUM_NEURONS), jnp.float32)],
    )(input_bits, connt)


@functools.lru_cache(maxsize=1)
def _make_gather():
    mesh = plsc.VectorSubcoreMesh(core_axis_name="c", subcore_axis_name="s")

    @functools.partial(
        pl.kernel,
        mesh=mesh,
        out_type=jax.ShapeDtypeStruct((_FLAT,), jnp.float32),
        scratch_types=[
            pltpu.VMEM((_CH,), jnp.int32),
            pltpu.VMEM((_CH,), jnp.int32),
            pltpu.VMEM((_CH,), jnp.float32),
            pltpu.VMEM((_CH,), jnp.float32),
            pltpu.SemaphoreType.DMA,
            pltpu.SemaphoreType.DMA,
            pltpu.SemaphoreType.DMA,
        ],
    )
    def gather_k(mem_hbm, idx_hbm, out_hbm, idx_v0, idx_v1, val_v0, val_v1,
                 gsem0, gsem1, wsem):
        wid = lax.axis_index("s") * 2 + lax.axis_index("c")
        base = wid * _PER_W
        idx_bufs = (idx_v0, idx_v1)
        val_bufs = (val_v0, val_v1)
        half = _CH // 2

        def start_gather(buf):
            g0 = pltpu.async_copy(
                mem_hbm.at[idx_bufs[buf].at[pl.ds(0, half)]],
                val_bufs[buf].at[pl.ds(0, half)], gsem0,
            )
            g1 = pltpu.async_copy(
                mem_hbm.at[idx_bufs[buf].at[pl.ds(half, half)]],
                val_bufs[buf].at[pl.ds(half, half)], gsem1,
            )
            return g0, g1

        pltpu.sync_copy(idx_hbm.at[pl.ds(base, _CH)], idx_v0)
        gathers = [None, None]
        wbs = [None, None]
        gathers[0] = start_gather(0)
        for c in range(_NCH):
            cur = c % 2
            nxt = (c + 1) % 2
            if c + 1 < _NCH:
                pltpu.sync_copy(
                    idx_hbm.at[pl.ds(base + (c + 1) * _CH, _CH)], idx_bufs[nxt]
                )
            gathers[cur][0].wait()
            gathers[cur][1].wait()
            wbs[cur] = pltpu.async_copy(
                val_bufs[cur], out_hbm.at[pl.ds(base + c * _CH, _CH)], wsem
            )
            if c + 1 < _NCH:
                if wbs[nxt] is not None:
                    wbs[nxt].wait()
                gathers[nxt] = start_gather(nxt)
        wbs[(_NCH - 1) % 2].wait()

    return gather_k


def kernel(input_bits, connections, memory):
    connt = jnp.full((16, _NUM_NEURONS), _TOTAL_BITS, jnp.int32)
    connt = connt.at[:_N_BITS].set(connections.T)
    idx = _addresses(input_bits, connt)
    mem_lin = (
        memory.reshape(_NUM_NEURONS // 8, 8, _MEM_SIZE // 128, 128)
        .transpose(0, 2, 1, 3)
        .reshape(_NUM_NEURONS * _MEM_SIZE)
    )
    out_flat = _make_gather()(mem_lin, idx)
    return (
        out_flat.reshape(_B // 8, 8, _NUM_NEURONS // 128, 128)
        .transpose(0, 2, 1, 3)
        .reshape(_B, _NUM_NEURONS)
    )


# speedup vs baseline: 3.5521x; 1.1614x over previous
import functools

import jax
import jax.numpy as jnp
from jax import lax
from jax.experimental import pallas as pl
from jax.experimental.pallas import tpu as pltpu
from jax.experimental.pallas import tpu_sc as plsc

_B = 4096
_TOTAL_BITS = 1024
_NUM_NEURONS = 1024
_N_BITS = 14
_MEM_SIZE = 1 << _N_BITS

_BM = 512
_NW = 32
_FLAT = _B * _NUM_NEURONS
_PER_W = _FLAT // _NW
_CH = 16384
_NCH = _PER_W // _CH


def _addr_body(bits_ref, connt_ref, out_ref, whi_ref, wlo_ref):
    @pl.when(pl.program_id(0) == 0)
    def _build_w():
        t = lax.broadcasted_iota(jnp.int32, (_TOTAL_BITS, _NUM_NEURONS), 0)
        hi = jnp.zeros((_TOTAL_BITS, _NUM_NEURONS), jnp.float32)
        lo = jnp.zeros((_TOTAL_BITS, _NUM_NEURONS), jnp.float32)
        for k in range(7):
            c = connt_ref[k : k + 1, :]
            hi = hi + jnp.where(t == c, jnp.float32(1 << (6 - k)), 0.0)
        for k in range(7, _N_BITS):
            c = connt_ref[k : k + 1, :]
            lo = lo + jnp.where(t == c, jnp.float32(1 << (_N_BITS - 1 - k)), 0.0)
        whi_ref[:] = hi.astype(jnp.bfloat16)
        wlo_ref[:] = lo.astype(jnp.bfloat16)

    bits = (bits_ref[:] != 0).astype(jnp.bfloat16)
    hi = jnp.dot(bits, whi_ref[:], preferred_element_type=jnp.float32)
    lo = jnp.dot(bits, wlo_ref[:], preferred_element_type=jnp.float32)
    addr = (hi * 128.0 + lo).astype(jnp.int32)
    n = lax.broadcasted_iota(jnp.int32, (_BM, _NUM_NEURONS), 1)
    phys = (
        (n >> 3) * (_MEM_SIZE * 8)
        + (addr >> 7) * 1024
        + (n & 7) * 128
        + (addr & 127)
    )
    out_ref[:] = (
        phys.reshape(_BM // 8, 8, _NUM_NEURONS // 128, 128)
        .transpose(0, 2, 1, 3)
        .reshape(_BM * _NUM_NEURONS)
    )


def _addresses(input_bits, connt):
    return pl.pallas_call(
        _addr_body,
        grid=(_B // _BM,),
        in_specs=[
            pl.BlockSpec((_BM, _TOTAL_BITS), lambda i: (i, 0)),
            pl.BlockSpec((16, _NUM_NEURONS), lambda i: (0, 0)),
        ],
        out_specs=pl.BlockSpec((_BM * _NUM_NEURONS,), lambda i: (i,)),
        out_shape=jax.ShapeDtypeStruct((_FLAT,), jnp.int32),
        scratch_shapes=[
            pltpu.VMEM((_TOTAL_BITS, _NUM_NEURONS), jnp.bfloat16),
            pltpu.VMEM((_TOTAL_BITS, _NUM_NEURONS), jnp.bfloat16),
        ],
    )(input_bits, connt)


@functools.lru_cache(maxsize=1)
def _make_gather():
    mesh = plsc.VectorSubcoreMesh(core_axis_name="c", subcore_axis_name="s")

    @functools.partial(
        pl.kernel,
        mesh=mesh,
        out_type=jax.ShapeDtypeStruct((_FLAT,), jnp.float32),
        scratch_types=[
            pltpu.VMEM((_CH,), jnp.int32),
            pltpu.VMEM((_CH,), jnp.int32),
            pltpu.VMEM((_CH,), jnp.float32),
            pltpu.VMEM((_CH,), jnp.float32),
            pltpu.SemaphoreType.DMA,
            pltpu.SemaphoreType.DMA,
            pltpu.SemaphoreType.DMA,
        ],
    )
    def gather_k(mem_hbm, idx_hbm, out_hbm, idx_v0, idx_v1, val_v0, val_v1,
                 gsem0, gsem1, wsem):
        wid = lax.axis_index("s") * 2 + lax.axis_index("c")
        base = wid * _PER_W
        idx_bufs = (idx_v0, idx_v1)
        val_bufs = (val_v0, val_v1)
        half = _CH // 2

        def start_gather(buf):
            g0 = pltpu.async_copy(
                mem_hbm.at[idx_bufs[buf].at[pl.ds(0, half)]],
                val_bufs[buf].at[pl.ds(0, half)], gsem0,
            )
            g1 = pltpu.async_copy(
                mem_hbm.at[idx_bufs[buf].at[pl.ds(half, half)]],
                val_bufs[buf].at[pl.ds(half, half)], gsem1,
            )
            return g0, g1

        pltpu.sync_copy(idx_hbm.at[pl.ds(base, _CH)], idx_v0)
        gathers = [None, None]
        wbs = [None, None]
        gathers[0] = start_gather(0)
        for c in range(_NCH):
            cur = c % 2
            nxt = (c + 1) % 2
            if c + 1 < _NCH:
                pltpu.sync_copy(
                    idx_hbm.at[pl.ds(base + (c + 1) * _CH, _CH)], idx_bufs[nxt]
                )
            gathers[cur][0].wait()
            gathers[cur][1].wait()
            wbs[cur] = pltpu.async_copy(
                val_bufs[cur], out_hbm.at[pl.ds(base + c * _CH, _CH)], wsem
            )
            if c + 1 < _NCH:
                if wbs[nxt] is not None:
                    wbs[nxt].wait()
                gathers[nxt] = start_gather(nxt)
        wbs[(_NCH - 1) % 2].wait()

    return gather_k


def kernel(input_bits, connections, memory):
    connt = jnp.full((16, _NUM_NEURONS), _TOTAL_BITS, jnp.int32)
    connt = connt.at[:_N_BITS].set(connections.T)
    idx = _addresses(input_bits, connt)
    mem_lin = (
        memory.reshape(_NUM_NEURONS // 8, 8, _MEM_SIZE // 128, 128)
        .transpose(0, 2, 1, 3)
        .reshape(_NUM_NEURONS * _MEM_SIZE)
    )
    out_flat = _make_gather()(mem_lin, idx)
    return (
        out_flat.reshape(_B // 8, 8, _NUM_NEURONS // 128, 128)
        .transpose(0, 2, 1, 3)
        .reshape(_B, _NUM_NEURONS)
    )


# speedup vs baseline: 3.6098x; 1.0162x over previous
import functools

import jax
import jax.numpy as jnp
from jax import lax
from jax.experimental import pallas as pl
from jax.experimental.pallas import tpu as pltpu
from jax.experimental.pallas import tpu_sc as plsc

_B = 4096
_TOTAL_BITS = 1024
_NUM_NEURONS = 1024
_N_BITS = 14
_MEM_SIZE = 1 << _N_BITS

_BM = 512
_NW = 32
_FLAT = _B * _NUM_NEURONS
_PER_W = _FLAT // _NW
_CH = 16384
_NCH = _PER_W // _CH


def _addr_body(bits_ref, connt_ref, out_ref, whi_ref, wlo_ref, nb_ref):
    @pl.when(pl.program_id(0) == 0)
    def _build_w():
        t = lax.broadcasted_iota(jnp.int32, (_TOTAL_BITS, _NUM_NEURONS), 0)
        hi = jnp.zeros((_TOTAL_BITS, _NUM_NEURONS), jnp.float32)
        lo = jnp.zeros((_TOTAL_BITS, _NUM_NEURONS), jnp.float32)
        for k in range(7):
            c = connt_ref[k : k + 1, :]
            hi = hi + jnp.where(t == c, jnp.float32(1024 << (6 - k)), 0.0)
        for k in range(7, _N_BITS):
            c = connt_ref[k : k + 1, :]
            lo = lo + jnp.where(t == c, jnp.float32(1 << (_N_BITS - 1 - k)), 0.0)
        whi_ref[:] = hi.astype(jnp.bfloat16)
        wlo_ref[:] = lo.astype(jnp.bfloat16)
        nn = lax.broadcasted_iota(jnp.int32, (8, _NUM_NEURONS), 1)
        nb_ref[:] = (nn >> 3) * (_MEM_SIZE * 8) + (nn & 7) * 128

    bits = (bits_ref[:] != 0).astype(jnp.bfloat16)
    hi = jnp.dot(bits, whi_ref[:], preferred_element_type=jnp.float32)
    lo = jnp.dot(bits, wlo_ref[:], preferred_element_type=jnp.float32)
    phys = (hi + lo).astype(jnp.int32) + nb_ref[0:1, :]
    out_ref[:] = (
        phys.reshape(_BM // 8, 8, _NUM_NEURONS // 128, 128)
        .transpose(0, 2, 1, 3)
        .reshape(_BM * _NUM_NEURONS)
    )


def _addresses(input_bits, connt):
    return pl.pallas_call(
        _addr_body,
        grid=(_B // _BM,),
        in_specs=[
            pl.BlockSpec((_BM, _TOTAL_BITS), lambda i: (i, 0)),
            pl.BlockSpec((16, _NUM_NEURONS), lambda i: (0, 0)),
        ],
        out_specs=pl.BlockSpec((_BM * _NUM_NEURONS,), lambda i: (i,)),
        out_shape=jax.ShapeDtypeStruct((_FLAT,), jnp.int32),
        scratch_shapes=[
            pltpu.VMEM((_TOTAL_BITS, _NUM_NEURONS), jnp.bfloat16),
            pltpu.VMEM((_TOTAL_BITS, _NUM_NEURONS), jnp.bfloat16),
            pltpu.VMEM((8, _NUM_NEURONS), jnp.int32),
        ],
    )(input_bits, connt)


@functools.lru_cache(maxsize=1)
def _make_gather():
    mesh = plsc.VectorSubcoreMesh(core_axis_name="c", subcore_axis_name="s")

    @functools.partial(
        pl.kernel,
        mesh=mesh,
        out_type=jax.ShapeDtypeStruct((_FLAT,), jnp.float32),
        scratch_types=[
            pltpu.VMEM((_CH,), jnp.int32),
            pltpu.VMEM((_CH,), jnp.int32),
            pltpu.VMEM((_CH,), jnp.float32),
            pltpu.VMEM((_CH,), jnp.float32),
            pltpu.SemaphoreType.DMA,
            pltpu.SemaphoreType.DMA,
            pltpu.SemaphoreType.DMA,
        ],
    )
    def gather_k(mem_hbm, idx_hbm, out_hbm, idx_v0, idx_v1, val_v0, val_v1,
                 gsem0, gsem1, wsem):
        wid = lax.axis_index("s") * 2 + lax.axis_index("c")
        base = wid * _PER_W
        idx_bufs = (idx_v0, idx_v1)
        val_bufs = (val_v0, val_v1)
        half = _CH // 2

        def start_gather(buf):
            g0 = pltpu.async_copy(
                mem_hbm.at[idx_bufs[buf].at[pl.ds(0, half)]],
                val_bufs[buf].at[pl.ds(0, half)], gsem0,
            )
            g1 = pltpu.async_copy(
                mem_hbm.at[idx_bufs[buf].at[pl.ds(half, half)]],
                val_bufs[buf].at[pl.ds(half, half)], gsem1,
            )
            return g0, g1

        pltpu.sync_copy(idx_hbm.at[pl.ds(base, _CH)], idx_v0)
        gathers = [None, None]
        wbs = [None, None]
        gathers[0] = start_gather(0)
        for c in range(_NCH):
            cur = c % 2
            nxt = (c + 1) % 2
            if c + 1 < _NCH:
                pltpu.sync_copy(
                    idx_hbm.at[pl.ds(base + (c + 1) * _CH, _CH)], idx_bufs[nxt]
                )
            gathers[cur][0].wait()
            gathers[cur][1].wait()
            wbs[cur] = pltpu.async_copy(
                val_bufs[cur], out_hbm.at[pl.ds(base + c * _CH, _CH)], wsem
            )
            if c + 1 < _NCH:
                if wbs[nxt] is not None:
                    wbs[nxt].wait()
                gathers[nxt] = start_gather(nxt)
        wbs[(_NCH - 1) % 2].wait()

    return gather_k


def kernel(input_bits, connections, memory):
    connt = jnp.full((16, _NUM_NEURONS), _TOTAL_BITS, jnp.int32)
    connt = connt.at[:_N_BITS].set(connections.T)
    idx = _addresses(input_bits, connt)
    mem_lin = (
        memory.reshape(_NUM_NEURONS // 8, 8, _MEM_SIZE // 128, 128)
        .transpose(0, 2, 1, 3)
        .reshape(_NUM_NEURONS * _MEM_SIZE)
    )
    out_flat = _make_gather()(mem_lin, idx)
    return (
        out_flat.reshape(_B // 8, 8, _NUM_NEURONS // 128, 128)
        .transpose(0, 2, 1, 3)
        .reshape(_B, _NUM_NEURONS)
    )
